# baseline (speedup 1.0000x reference)
import functools

import jax
import jax.numpy as jnp
from jax import lax
from jax.experimental import pallas as pl
from jax.experimental.pallas import tpu as pltpu
from jax.experimental.pallas import tpu_sc as plsc

D = 64
D_IN = 75
DH = 32
N = 50000
E = 800000
BATCH = 100

_NTILES = 16
_EPT = E // _NTILES
_CHUNK = 400
_NFULL = _EPT // _CHUNK
_NPT = 3128
_NPT_LAST = N - _NPT * (_NTILES - 1)


def _segsum_body(hlo, hhi, src, dst, zeros, out_lo, out_hi,
                 idx_v, dst_v, rows_v, acc, sem):
    c = lax.axis_index("c")
    s = lax.axis_index("s")
    @pl.when(s < _NTILES - 1)
    def _():
        pltpu.sync_copy(zeros.at[pl.ds(s * _NPT, _NPT)], acc.at[pl.ds(s * _NPT, _NPT)])

    @pl.when(s == _NTILES - 1)
    def _():
        pltpu.sync_copy(zeros.at[pl.ds(s * _NPT, _NPT_LAST)],
                        acc.at[pl.ds(s * _NPT, _NPT_LAST)])

    plsc.subcore_barrier()

    ebase = s * _EPT

    def chunk(off, idx_ref, dst_ref, rows_ref, count):
        pltpu.sync_copy(src.at[pl.ds(off, count)], idx_ref)
        pltpu.sync_copy(dst.at[pl.ds(off, count)], dst_ref)

        @pl.when(c == 0)
        def _():
            pltpu.async_copy(hlo.at[idx_ref], rows_ref, sem).wait()

        @pl.when(c == 1)
        def _():
            pltpu.async_copy(hhi.at[idx_ref], rows_ref, sem).wait()

        pltpu.sync_copy(rows_ref, acc.at[dst_ref], add=True)

    def body(g, carry):
        chunk(ebase + g * _CHUNK, idx_v, dst_v, rows_v, _CHUNK)
        return carry

    lax.fori_loop(0, _NFULL, body, 0)

    plsc.subcore_barrier()

    def writeout(out_ref):
        @pl.when(s < _NTILES - 1)
        def _():
            pltpu.sync_copy(acc.at[pl.ds(s * _NPT, _NPT)],
                            out_ref.at[pl.ds(s * _NPT, _NPT)])

        @pl.when(s == _NTILES - 1)
        def _():
            pltpu.sync_copy(acc.at[pl.ds(s * _NPT, _NPT_LAST)],
                            out_ref.at[pl.ds(s * _NPT, _NPT_LAST)])

    @pl.when(c == 0)
    def _():
        writeout(out_lo)

    @pl.when(c == 1)
    def _():
        writeout(out_hi)


_segsum_call = functools.partial(
    pl.kernel,
    mesh=plsc.VectorSubcoreMesh(core_axis_name="c", subcore_axis_name="s"),
    compiler_params=pltpu.CompilerParams(use_tc_tiling_on_sc=False),
    out_type=[jax.ShapeDtypeStruct((N, DH), jnp.float32),
              jax.ShapeDtypeStruct((N, DH), jnp.float32)],
    scratch_types=[
        pltpu.VMEM((_CHUNK,), jnp.int32),
        pltpu.VMEM((_CHUNK,), jnp.int32),
        pltpu.VMEM((_CHUNK, DH), jnp.float32),
        pltpu.VMEM_SHARED((N, DH), jnp.float32),
        pltpu.SemaphoreType.DMA,
    ],
)(_segsum_body)


def _graph_sum_sc(src, dst, h):
    h_lo = h[:, :DH]
    h_hi = h[:, DH:]
    zeros = jnp.zeros((N, DH), jnp.float32)
    out_lo, out_hi = _segsum_call(h_lo, h_hi, src, dst, zeros)
    return jnp.concatenate([out_lo, out_hi], axis=1)


_BLK = 2000
_NB = N // _BLK


def _gcn_dense_kernel(agg_ref, ft_ref, w_ref, b_ref, wt_ref, br_ref,
                      g_ref, be_ref, o_ref, s_ref, q_ref):
    p = pl.program_id(0)
    y = jax.nn.relu(jnp.dot(agg_ref[...], w_ref[...],
                            preferred_element_type=jnp.float32) + b_ref[...])
    y = y + jax.nn.relu(jnp.dot(ft_ref[...], wt_ref[...],
                                preferred_element_type=jnp.float32) + br_ref[...])

    @pl.when(jnp.logical_and(p == 0, pl.program_id(1) == 0))
    def _():
        s_ref[...] = jnp.zeros_like(s_ref)
        q_ref[...] = jnp.zeros_like(q_ref)

    @pl.when(p == 0)
    def _():
        s_ref[...] += jnp.sum(y, axis=0, keepdims=True)
        q_ref[...] += jnp.sum(y * y, axis=0, keepdims=True)

    @pl.when(p == 1)
    def _():
        mu = s_ref[...] / N
        var = q_ref[...] / N - mu * mu
        o_ref[...] = (y - mu) / jnp.sqrt(var + 1e-5) * g_ref[...] + be_ref[...]


def _gcn_dense(agg, feats, p):
    row = lambda ph, j: (j, 0)
    mat = lambda ph, j: (0, 0)
    return pl.pallas_call(
        _gcn_dense_kernel,
        grid=(2, _NB),
        in_specs=[
            pl.BlockSpec((_BLK, D), row),
            pl.BlockSpec((_BLK, D), row),
            pl.BlockSpec((D, D), mat),
            pl.BlockSpec((1, D), mat),
            pl.BlockSpec((D, D), mat),
            pl.BlockSpec((1, D), mat),
            pl.BlockSpec((1, D), mat),
            pl.BlockSpec((1, D), mat),
        ],
        out_specs=pl.BlockSpec((_BLK, D), row),
        scratch_shapes=[pltpu.VMEM((1, D), jnp.float32),
                        pltpu.VMEM((1, D), jnp.float32)],
        out_shape=jax.ShapeDtypeStruct((N, D), jnp.float32),
    )(agg, feats, p['W'], p['b'].reshape(1, D), p['Wr'].T,
      p['br'].reshape(1, D), p['g'].reshape(1, D), p['be'].reshape(1, D))


def _make_stem_kernel(eps, add_res):
    def body(x_ref, wt_ref, b_ref, g_ref, be_ref, o_ref, s_ref, q_ref):
        p = pl.program_id(0)
        y = jnp.dot(x_ref[...], wt_ref[...],
                    preferred_element_type=jnp.float32) + b_ref[...]
        y = 0.5 * y * (1.0 + lax.erf(y * 0.7071067811865476))

        @pl.when(jnp.logical_and(p == 0, pl.program_id(1) == 0))
        def _():
            s_ref[...] = jnp.zeros_like(s_ref)
            q_ref[...] = jnp.zeros_like(q_ref)

        @pl.when(p == 0)
        def _():
            s_ref[...] += jnp.sum(y, axis=0, keepdims=True)
            q_ref[...] += jnp.sum(y * y, axis=0, keepdims=True)

        @pl.when(p == 1)
        def _():
            mu = s_ref[...] / N
            var = q_ref[...] / N - mu * mu
            o = (y - mu) / jnp.sqrt(var + eps) * g_ref[...] + be_ref[...]
            if add_res:
                o = o + y
            o_ref[...] = o

    return body


def _stem_layer(x, w, b, g, be, eps, add_res):
    row = lambda ph, j: (j, 0)
    mat = lambda ph, j: (0, 0)
    return pl.pallas_call(
        _make_stem_kernel(eps, add_res),
        grid=(2, _NB),
        in_specs=[
            pl.BlockSpec((_BLK, D), row),
            pl.BlockSpec((D, D), mat),
            pl.BlockSpec((1, D), mat),
            pl.BlockSpec((1, D), mat),
            pl.BlockSpec((1, D), mat),
        ],
        out_specs=pl.BlockSpec((_BLK, D), row),
        scratch_shapes=[pltpu.VMEM((1, D), jnp.float32),
                        pltpu.VMEM((1, D), jnp.float32)],
        out_shape=jax.ShapeDtypeStruct((N, D), jnp.float32),
    )(x, w.T, b.reshape(1, D), g.reshape(1, D), be.reshape(1, D))


def _init_kernel(x_ref, wt_ref, o_ref):
    o_ref[...] = jnp.dot(x_ref[...], wt_ref[...],
                         preferred_element_type=jnp.float32)


def _init_transform(x, w):
    return pl.pallas_call(
        _init_kernel,
        grid=(_NB,),
        in_specs=[
            pl.BlockSpec((_BLK, D_IN), lambda j: (j, 0)),
            pl.BlockSpec((D_IN, D), lambda j: (0, 0)),
        ],
        out_specs=pl.BlockSpec((_BLK, D), lambda j: (j, 0)),
        out_shape=jax.ShapeDtypeStruct((N, D), jnp.float32),
    )(x, w.T)


def _graph_sum(src, dst, h, n):
    return _graph_sum_sc(src, dst, h)


def _graph_max(src, dst, h, n):
    m = jax.ops.segment_max(h[src], dst, num_segments=n)
    return jnp.where(jnp.isfinite(m), m, 0.0)


def _gcn_layer(p, src, dst, n, feats):
    agg = _graph_sum(src, dst, feats, n)
    return _gcn_dense(agg, feats, p)


def kernel(x, edge_index, params, batch_size):
    src = edge_index[0]
    dst = edge_index[1]
    n = x.shape[0]
    h = _init_transform(x, params['W_init'])
    h = _stem_layer(h, params['s_W1'], params['s_b1'], params['s_g1'],
                    params['s_be1'], 0.1, False)
    h = _stem_layer(h, params['s_W2'], params['s_b2'], params['s_g2'],
                    params['s_be2'], 0.1, True)
    result = jnp.zeros_like(h)
    for i in range(2):
        h = _gcn_layer(params['gnn'][i], src, dst, n, h)
        pooled = _graph_max(src, dst, h, n)
        result = result + _gcn_layer(params['smooth'][i], src, dst, n, pooled)
    result = _gcn_layer(params['smooth_final'], src, dst, n, result)
    return result.reshape(BATCH, -1, D)


# speedup vs baseline: 1.0723x; 1.0723x over previous
import functools

import jax
import jax.numpy as jnp
from jax import lax
from jax.experimental import pallas as pl
from jax.experimental.pallas import tpu as pltpu
from jax.experimental.pallas import tpu_sc as plsc

D = 64
D_IN = 75
DH = 32
N = 50000
E = 800000
BATCH = 100

_NTILES = 16
_EPT = E // _NTILES
_CHUNK = 400
_NFULL = _EPT // _CHUNK
_NPT = 3128
_NPT_LAST = N - _NPT * (_NTILES - 1)


def _segsum_body(hlo, hhi, src, dst, zeros, out_lo, out_hi,
                 idx_v, dst_v, rows_v, idx_b, dst_b, rows_b, acc, sem, sem_b):
    c = lax.axis_index("c")
    s = lax.axis_index("s")
    @pl.when(s < _NTILES - 1)
    def _():
        pltpu.sync_copy(zeros.at[pl.ds(s * _NPT, _NPT)], acc.at[pl.ds(s * _NPT, _NPT)])

    @pl.when(s == _NTILES - 1)
    def _():
        pltpu.sync_copy(zeros.at[pl.ds(s * _NPT, _NPT_LAST)],
                        acc.at[pl.ds(s * _NPT, _NPT_LAST)])

    plsc.subcore_barrier()

    ebase = s * _EPT

    def load_idx(g, islot, dslot):
        off = ebase + g * _CHUNK
        pltpu.sync_copy(src.at[pl.ds(off, _CHUNK)], islot)
        pltpu.sync_copy(dst.at[pl.ds(off, _CHUNK)], dslot)

    def start_gather(islot, rslot, sem_g):
        @pl.when(c == 0)
        def _():
            pltpu.async_copy(hlo.at[islot], rslot, sem_g)

        @pl.when(c == 1)
        def _():
            pltpu.async_copy(hhi.at[islot], rslot, sem_g)

    def wait_gather(islot, rslot, sem_g):
        @pl.when(c == 0)
        def _():
            pltpu.make_async_copy(hlo.at[islot], rslot, sem_g).wait()

        @pl.when(c == 1)
        def _():
            pltpu.make_async_copy(hhi.at[islot], rslot, sem_g).wait()

    load_idx(0, idx_v, dst_v)
    start_gather(idx_v, rows_v, sem)
    load_idx(1, idx_b, dst_b)
    start_gather(idx_b, rows_b, sem_b)

    def pair(gg, carry):
        g0 = gg * 2

        wait_gather(idx_v, rows_v, sem)
        pltpu.sync_copy(rows_v, acc.at[dst_v], add=True)

        @pl.when(g0 + 2 < _NFULL)
        def _():
            load_idx(g0 + 2, idx_v, dst_v)
            start_gather(idx_v, rows_v, sem)

        wait_gather(idx_b, rows_b, sem_b)
        pltpu.sync_copy(rows_b, acc.at[dst_b], add=True)

        @pl.when(g0 + 3 < _NFULL)
        def _():
            load_idx(g0 + 3, idx_b, dst_b)
            start_gather(idx_b, rows_b, sem_b)

        return carry

    lax.fori_loop(0, _NFULL // 2, pair, 0)

    @pl.when(_NFULL % 2 == 1)
    def _():
        wait_gather(idx_v, rows_v, sem)
        pltpu.sync_copy(rows_v, acc.at[dst_v], add=True)

        plsc.subcore_barrier()

    def writeout(out_ref):
        @pl.when(s < _NTILES - 1)
        def _():
            pltpu.sync_copy(acc.at[pl.ds(s * _NPT, _NPT)],
                            out_ref.at[pl.ds(s * _NPT, _NPT)])

        @pl.when(s == _NTILES - 1)
        def _():
            pltpu.sync_copy(acc.at[pl.ds(s * _NPT, _NPT_LAST)],
                            out_ref.at[pl.ds(s * _NPT, _NPT_LAST)])

    @pl.when(c == 0)
    def _():
        writeout(out_lo)

    @pl.when(c == 1)
    def _():
        writeout(out_hi)


_segsum_call = functools.partial(
    pl.kernel,
    mesh=plsc.VectorSubcoreMesh(core_axis_name="c", subcore_axis_name="s"),
    compiler_params=pltpu.CompilerParams(use_tc_tiling_on_sc=False),
    out_type=[jax.ShapeDtypeStruct((N, DH), jnp.float32),
              jax.ShapeDtypeStruct((N, DH), jnp.float32)],
    scratch_types=[
        pltpu.VMEM((_CHUNK,), jnp.int32),
        pltpu.VMEM((_CHUNK,), jnp.int32),
        pltpu.VMEM((_CHUNK, DH), jnp.float32),
        pltpu.VMEM((_CHUNK,), jnp.int32),
        pltpu.VMEM((_CHUNK,), jnp.int32),
        pltpu.VMEM((_CHUNK, DH), jnp.float32),
        pltpu.VMEM_SHARED((N, DH), jnp.float32),
        pltpu.SemaphoreType.DMA,
        pltpu.SemaphoreType.DMA,
    ],
)(_segsum_body)


def _graph_sum_sc(src, dst, h):
    h_lo = h[:, :DH]
    h_hi = h[:, DH:]
    zeros = jnp.zeros((N, DH), jnp.float32)
    out_lo, out_hi = _segsum_call(h_lo, h_hi, src, dst, zeros)
    return jnp.concatenate([out_lo, out_hi], axis=1)


_BLK = 2000
_NB = N // _BLK


def _gcn_dense_kernel(agg_ref, ft_ref, w_ref, b_ref, wt_ref, br_ref,
                      g_ref, be_ref, o_ref, s_ref, q_ref):
    p = pl.program_id(0)
    y = jax.nn.relu(jnp.dot(agg_ref[...], w_ref[...],
                            preferred_element_type=jnp.float32) + b_ref[...])
    y = y + jax.nn.relu(jnp.dot(ft_ref[...], wt_ref[...],
                                preferred_element_type=jnp.float32) + br_ref[...])

    @pl.when(jnp.logical_and(p == 0, pl.program_id(1) == 0))
    def _():
        s_ref[...] = jnp.zeros_like(s_ref)
        q_ref[...] = jnp.zeros_like(q_ref)

    @pl.when(p == 0)
    def _():
        s_ref[...] += jnp.sum(y, axis=0, keepdims=True)
        q_ref[...] += jnp.sum(y * y, axis=0, keepdims=True)

    @pl.when(p == 1)
    def _():
        mu = s_ref[...] / N
        var = q_ref[...] / N - mu * mu
        o_ref[...] = (y - mu) / jnp.sqrt(var + 1e-5) * g_ref[...] + be_ref[...]


def _gcn_dense(agg, feats, p):
    row = lambda ph, j: (j, 0)
    mat = lambda ph, j: (0, 0)
    return pl.pallas_call(
        _gcn_dense_kernel,
        grid=(2, _NB),
        in_specs=[
            pl.BlockSpec((_BLK, D), row),
            pl.BlockSpec((_BLK, D), row),
            pl.BlockSpec((D, D), mat),
            pl.BlockSpec((1, D), mat),
            pl.BlockSpec((D, D), mat),
            pl.BlockSpec((1, D), mat),
            pl.BlockSpec((1, D), mat),
            pl.BlockSpec((1, D), mat),
        ],
        out_specs=pl.BlockSpec((_BLK, D), row),
        scratch_shapes=[pltpu.VMEM((1, D), jnp.float32),
                        pltpu.VMEM((1, D), jnp.float32)],
        out_shape=jax.ShapeDtypeStruct((N, D), jnp.float32),
    )(agg, feats, p['W'], p['b'].reshape(1, D), p['Wr'].T,
      p['br'].reshape(1, D), p['g'].reshape(1, D), p['be'].reshape(1, D))


def _make_stem_kernel(eps, add_res):
    def body(x_ref, wt_ref, b_ref, g_ref, be_ref, o_ref, s_ref, q_ref):
        p = pl.program_id(0)
        y = jnp.dot(x_ref[...], wt_ref[...],
                    preferred_element_type=jnp.float32) + b_ref[...]
        y = 0.5 * y * (1.0 + lax.erf(y * 0.7071067811865476))

        @pl.when(jnp.logical_and(p == 0, pl.program_id(1) == 0))
        def _():
            s_ref[...] = jnp.zeros_like(s_ref)
            q_ref[...] = jnp.zeros_like(q_ref)

        @pl.when(p == 0)
        def _():
            s_ref[...] += jnp.sum(y, axis=0, keepdims=True)
            q_ref[...] += jnp.sum(y * y, axis=0, keepdims=True)

        @pl.when(p == 1)
        def _():
            mu = s_ref[...] / N
            var = q_ref[...] / N - mu * mu
            o = (y - mu) / jnp.sqrt(var + eps) * g_ref[...] + be_ref[...]
            if add_res:
                o = o + y
            o_ref[...] = o

    return body


def _stem_layer(x, w, b, g, be, eps, add_res):
    row = lambda ph, j: (j, 0)
    mat = lambda ph, j: (0, 0)
    return pl.pallas_call(
        _make_stem_kernel(eps, add_res),
        grid=(2, _NB),
        in_specs=[
            pl.BlockSpec((_BLK, D), row),
            pl.BlockSpec((D, D), mat),
            pl.BlockSpec((1, D), mat),
            pl.BlockSpec((1, D), mat),
            pl.BlockSpec((1, D), mat),
        ],
        out_specs=pl.BlockSpec((_BLK, D), row),
        scratch_shapes=[pltpu.VMEM((1, D), jnp.float32),
                        pltpu.VMEM((1, D), jnp.float32)],
        out_shape=jax.ShapeDtypeStruct((N, D), jnp.float32),
    )(x, w.T, b.reshape(1, D), g.reshape(1, D), be.reshape(1, D))


def _init_kernel(x_ref, wt_ref, o_ref):
    o_ref[...] = jnp.dot(x_ref[...], wt_ref[...],
                         preferred_element_type=jnp.float32)


def _init_transform(x, w):
    return pl.pallas_call(
        _init_kernel,
        grid=(_NB,),
        in_specs=[
            pl.BlockSpec((_BLK, D_IN), lambda j: (j, 0)),
            pl.BlockSpec((D_IN, D), lambda j: (0, 0)),
        ],
        out_specs=pl.BlockSpec((_BLK, D), lambda j: (j, 0)),
        out_shape=jax.ShapeDtypeStruct((N, D), jnp.float32),
    )(x, w.T)


def _graph_sum(src, dst, h, n):
    return _graph_sum_sc(src, dst, h)


def _graph_max(src, dst, h, n):
    m = jax.ops.segment_max(h[src], dst, num_segments=n)
    return jnp.where(jnp.isfinite(m), m, 0.0)


def _gcn_layer(p, src, dst, n, feats):
    agg = _graph_sum(src, dst, feats, n)
    return _gcn_dense(agg, feats, p)


def kernel(x, edge_index, params, batch_size):
    src = edge_index[0]
    dst = edge_index[1]
    n = x.shape[0]
    h = _init_transform(x, params['W_init'])
    h = _stem_layer(h, params['s_W1'], params['s_b1'], params['s_g1'],
                    params['s_be1'], 0.1, False)
    h = _stem_layer(h, params['s_W2'], params['s_b2'], params['s_g2'],
                    params['s_be2'], 0.1, True)
    result = jnp.zeros_like(h)
    for i in range(2):
        h = _gcn_layer(params['gnn'][i], src, dst, n, h)
        pooled = _graph_max(src, dst, h, n)
        result = result + _gcn_layer(params['smooth'][i], src, dst, n, pooled)
    result = _gcn_layer(params['smooth_final'], src, dst, n, result)
    return result.reshape(BATCH, -1, D)


# speedup vs baseline: 1.1028x; 1.0284x over previous
import functools

import jax
import jax.numpy as jnp
from jax import lax
from jax.experimental import pallas as pl
from jax.experimental.pallas import tpu as pltpu
from jax.experimental.pallas import tpu_sc as plsc

D = 64
D_IN = 75
DH = 32
N = 50000
E = 800000
BATCH = 100

_NTILES = 16
_EPT = E // _NTILES
_CHUNK = 400
_NFULL = _EPT // _CHUNK
_NPT = 3128
_NPT_LAST = N - _NPT * (_NTILES - 1)


def _segsum_body(hlo, hhi, src, dst, zeros, out_lo, out_hi,
                 idx_v, dst_v, rows_v, idx_b, dst_b, rows_b, acc, sem, sem_b,
                 sem_i, sem_ib):
    c = lax.axis_index("c")
    s = lax.axis_index("s")
    @pl.when(s < _NTILES - 1)
    def _():
        pltpu.sync_copy(zeros.at[pl.ds(s * _NPT, _NPT)], acc.at[pl.ds(s * _NPT, _NPT)])

    @pl.when(s == _NTILES - 1)
    def _():
        pltpu.sync_copy(zeros.at[pl.ds(s * _NPT, _NPT_LAST)],
                        acc.at[pl.ds(s * _NPT, _NPT_LAST)])

    plsc.subcore_barrier()

    ebase = s * _EPT

    def load_idx(g, islot, dslot):
        off = ebase + g * _CHUNK
        pltpu.sync_copy(src.at[pl.ds(off, _CHUNK)], islot)
        pltpu.sync_copy(dst.at[pl.ds(off, _CHUNK)], dslot)

    def start_load_idx(g, islot, dslot, sem_i):
        off = ebase + g * _CHUNK
        pltpu.async_copy(src.at[pl.ds(off, _CHUNK)], islot, sem_i)
        pltpu.async_copy(dst.at[pl.ds(off, _CHUNK)], dslot, sem_i)

    def wait_load_idx(g, islot, dslot, sem_i):
        off = ebase + g * _CHUNK
        pltpu.make_async_copy(src.at[pl.ds(off, _CHUNK)], islot, sem_i).wait()
        pltpu.make_async_copy(dst.at[pl.ds(off, _CHUNK)], dslot, sem_i).wait()

    def start_gather(islot, rslot, sem_g):
        @pl.when(c == 0)
        def _():
            pltpu.async_copy(hlo.at[islot], rslot, sem_g)

        @pl.when(c == 1)
        def _():
            pltpu.async_copy(hhi.at[islot], rslot, sem_g)

    def wait_gather(islot, rslot, sem_g):
        @pl.when(c == 0)
        def _():
            pltpu.make_async_copy(hlo.at[islot], rslot, sem_g).wait()

        @pl.when(c == 1)
        def _():
            pltpu.make_async_copy(hhi.at[islot], rslot, sem_g).wait()

    load_idx(0, idx_v, dst_v)
    start_gather(idx_v, rows_v, sem)
    load_idx(1, idx_b, dst_b)
    start_gather(idx_b, rows_b, sem_b)

    def pair(gg, carry):
        g0 = gg * 2

        wait_gather(idx_v, rows_v, sem)
        pltpu.sync_copy(rows_v, acc.at[dst_v], add=True)

        @pl.when(g0 + 2 < _NFULL)
        def _():
            start_load_idx(g0 + 2, idx_v, dst_v, sem_i)

        wait_gather(idx_b, rows_b, sem_b)

        @pl.when(g0 + 2 < _NFULL)
        def _():
            wait_load_idx(g0 + 2, idx_v, dst_v, sem_i)
            start_gather(idx_v, rows_v, sem)

        pltpu.sync_copy(rows_b, acc.at[dst_b], add=True)

        @pl.when(g0 + 3 < _NFULL)
        def _():
            start_load_idx(g0 + 3, idx_b, dst_b, sem_ib)
            wait_load_idx(g0 + 3, idx_b, dst_b, sem_ib)
            start_gather(idx_b, rows_b, sem_b)

        return carry

    lax.fori_loop(0, _NFULL // 2, pair, 0)

    @pl.when(_NFULL % 2 == 1)
    def _():
        wait_gather(idx_v, rows_v, sem)
        pltpu.sync_copy(rows_v, acc.at[dst_v], add=True)

        plsc.subcore_barrier()

    def writeout(out_ref):
        @pl.when(s < _NTILES - 1)
        def _():
            pltpu.sync_copy(acc.at[pl.ds(s * _NPT, _NPT)],
                            out_ref.at[pl.ds(s * _NPT, _NPT)])

        @pl.when(s == _NTILES - 1)
        def _():
            pltpu.sync_copy(acc.at[pl.ds(s * _NPT, _NPT_LAST)],
                            out_ref.at[pl.ds(s * _NPT, _NPT_LAST)])

    @pl.when(c == 0)
    def _():
        writeout(out_lo)

    @pl.when(c == 1)
    def _():
        writeout(out_hi)


_segsum_call = functools.partial(
    pl.kernel,
    mesh=plsc.VectorSubcoreMesh(core_axis_name="c", subcore_axis_name="s"),
    compiler_params=pltpu.CompilerParams(use_tc_tiling_on_sc=False),
    out_type=[jax.ShapeDtypeStruct((N, DH), jnp.float32),
              jax.ShapeDtypeStruct((N, DH), jnp.float32)],
    scratch_types=[
        pltpu.VMEM((_CHUNK,), jnp.int32),
        pltpu.VMEM((_CHUNK,), jnp.int32),
        pltpu.VMEM((_CHUNK, DH), jnp.float32),
        pltpu.VMEM((_CHUNK,), jnp.int32),
        pltpu.VMEM((_CHUNK,), jnp.int32),
        pltpu.VMEM((_CHUNK, DH), jnp.float32),
        pltpu.VMEM_SHARED((N, DH), jnp.float32),
        pltpu.SemaphoreType.DMA,
        pltpu.SemaphoreType.DMA,
        pltpu.SemaphoreType.DMA,
        pltpu.SemaphoreType.DMA,
    ],
)(_segsum_body)


def _graph_sum_sc(src, dst, h):
    h_lo = h[:, :DH]
    h_hi = h[:, DH:]
    zeros = jnp.zeros((N, DH), jnp.float32)
    out_lo, out_hi = _segsum_call(h_lo, h_hi, src, dst, zeros)
    return jnp.concatenate([out_lo, out_hi], axis=1)


_BLK = 2000
_NB = N // _BLK


def _gcn_dense_kernel(agg_ref, ft_ref, w_ref, b_ref, wt_ref, br_ref,
                      g_ref, be_ref, o_ref, s_ref, q_ref):
    p = pl.program_id(0)
    y = jax.nn.relu(jnp.dot(agg_ref[...], w_ref[...],
                            preferred_element_type=jnp.float32) + b_ref[...])
    y = y + jax.nn.relu(jnp.dot(ft_ref[...], wt_ref[...],
                                preferred_element_type=jnp.float32) + br_ref[...])

    @pl.when(jnp.logical_and(p == 0, pl.program_id(1) == 0))
    def _():
        s_ref[...] = jnp.zeros_like(s_ref)
        q_ref[...] = jnp.zeros_like(q_ref)

    @pl.when(p == 0)
    def _():
        s_ref[...] += jnp.sum(y, axis=0, keepdims=True)
        q_ref[...] += jnp.sum(y * y, axis=0, keepdims=True)

    @pl.when(p == 1)
    def _():
        mu = s_ref[...] / N
        var = q_ref[...] / N - mu * mu
        o_ref[...] = (y - mu) / jnp.sqrt(var + 1e-5) * g_ref[...] + be_ref[...]


def _gcn_dense(agg, feats, p):
    row = lambda ph, j: (j, 0)
    mat = lambda ph, j: (0, 0)
    return pl.pallas_call(
        _gcn_dense_kernel,
        grid=(2, _NB),
        in_specs=[
            pl.BlockSpec((_BLK, D), row),
            pl.BlockSpec((_BLK, D), row),
            pl.BlockSpec((D, D), mat),
            pl.BlockSpec((1, D), mat),
            pl.BlockSpec((D, D), mat),
            pl.BlockSpec((1, D), mat),
            pl.BlockSpec((1, D), mat),
            pl.BlockSpec((1, D), mat),
        ],
        out_specs=pl.BlockSpec((_BLK, D), row),
        scratch_shapes=[pltpu.VMEM((1, D), jnp.float32),
                        pltpu.VMEM((1, D), jnp.float32)],
        out_shape=jax.ShapeDtypeStruct((N, D), jnp.float32),
    )(agg, feats, p['W'], p['b'].reshape(1, D), p['Wr'].T,
      p['br'].reshape(1, D), p['g'].reshape(1, D), p['be'].reshape(1, D))


def _make_stem_kernel(eps, add_res):
    def body(x_ref, wt_ref, b_ref, g_ref, be_ref, o_ref, s_ref, q_ref):
        p = pl.program_id(0)
        y = jnp.dot(x_ref[...], wt_ref[...],
                    preferred_element_type=jnp.float32) + b_ref[...]
        y = 0.5 * y * (1.0 + lax.erf(y * 0.7071067811865476))

        @pl.when(jnp.logical_and(p == 0, pl.program_id(1) == 0))
        def _():
            s_ref[...] = jnp.zeros_like(s_ref)
            q_ref[...] = jnp.zeros_like(q_ref)

        @pl.when(p == 0)
        def _():
            s_ref[...] += jnp.sum(y, axis=0, keepdims=True)
            q_ref[...] += jnp.sum(y * y, axis=0, keepdims=True)

        @pl.when(p == 1)
        def _():
            mu = s_ref[...] / N
            var = q_ref[...] / N - mu * mu
            o = (y - mu) / jnp.sqrt(var + eps) * g_ref[...] + be_ref[...]
            if add_res:
                o = o + y
            o_ref[...] = o

    return body


def _stem_layer(x, w, b, g, be, eps, add_res):
    row = lambda ph, j: (j, 0)
    mat = lambda ph, j: (0, 0)
    return pl.pallas_call(
        _make_stem_kernel(eps, add_res),
        grid=(2, _NB),
        in_specs=[
            pl.BlockSpec((_BLK, D), row),
            pl.BlockSpec((D, D), mat),
            pl.BlockSpec((1, D), mat),
            pl.BlockSpec((1, D), mat),
            pl.BlockSpec((1, D), mat),
        ],
        out_specs=pl.BlockSpec((_BLK, D), row),
        scratch_shapes=[pltpu.VMEM((1, D), jnp.float32),
                        pltpu.VMEM((1, D), jnp.float32)],
        out_shape=jax.ShapeDtypeStruct((N, D), jnp.float32),
    )(x, w.T, b.reshape(1, D), g.reshape(1, D), be.reshape(1, D))


def _init_kernel(x_ref, wt_ref, o_ref):
    o_ref[...] = jnp.dot(x_ref[...], wt_ref[...],
                         preferred_element_type=jnp.float32)


def _init_transform(x, w):
    return pl.pallas_call(
        _init_kernel,
        grid=(_NB,),
        in_specs=[
            pl.BlockSpec((_BLK, D_IN), lambda j: (j, 0)),
            pl.BlockSpec((D_IN, D), lambda j: (0, 0)),
        ],
        out_specs=pl.BlockSpec((_BLK, D), lambda j: (j, 0)),
        out_shape=jax.ShapeDtypeStruct((N, D), jnp.float32),
    )(x, w.T)


def _graph_sum(src, dst, h, n):
    return _graph_sum_sc(src, dst, h)


def _graph_max(src, dst, h, n):
    m = jax.ops.segment_max(h[src], dst, num_segments=n)
    return jnp.where(jnp.isfinite(m), m, 0.0)


def _gcn_layer(p, src, dst, n, feats):
    agg = _graph_sum(src, dst, feats, n)
    return _gcn_dense(agg, feats, p)


def kernel(x, edge_index, params, batch_size):
    src = edge_index[0]
    dst = edge_index[1]
    n = x.shape[0]
    h = _init_transform(x, params['W_init'])
    h = _stem_layer(h, params['s_W1'], params['s_b1'], params['s_g1'],
                    params['s_be1'], 0.1, False)
    h = _stem_layer(h, params['s_W2'], params['s_b2'], params['s_g2'],
                    params['s_be2'], 0.1, True)
    result = jnp.zeros_like(h)
    for i in range(2):
        h = _gcn_layer(params['gnn'][i], src, dst, n, h)
        pooled = _graph_max(src, dst, h, n)
        result = result + _gcn_layer(params['smooth'][i], src, dst, n, pooled)
    result = _gcn_layer(params['smooth_final'], src, dst, n, result)
    return result.reshape(BATCH, -1, D)
